# baseline (speedup 1.0000x reference)
import functools

import jax
import jax.numpy as jnp
from jax import lax
from jax.experimental import pallas as pl
from jax.experimental.pallas import tpu as pltpu

_W = 24


def _body(idx_ref, cqt_ref, pos_ref, out_ref, sems):
    B, C, F, T = cqt_ref.shape

    def issue(b):
        sel = idx_ref[b]
        for c in range(C):
            sem = sems.at[c]

            @pl.when(sel == c)
            def _():
                pltpu.make_async_copy(pos_ref.at[b, c], out_ref.at[b, c], sem).start()

            @pl.when(sel != c)
            def _():
                pltpu.make_async_copy(cqt_ref.at[b, c], out_ref.at[b, c], sem).start()

    def drain(b):
        for c in range(C):
            pltpu.make_async_copy(cqt_ref.at[b, c], out_ref.at[b, c], sems.at[c]).wait()

    def step(b, _):
        issue(b)

        @pl.when(b >= _W)
        def _():
            drain(b - _W)

        return 0

    lax.fori_loop(0, B, step, 0)

    def tail(j, _):
        drain(B - _W + j)
        return 0

    lax.fori_loop(0, _W, tail, 0)


def kernel(cqt, pos_cqt, channel_idx):
    B, C, F, T = cqt.shape
    idx = channel_idx.astype(jnp.int32)

    grid_spec = pltpu.PrefetchScalarGridSpec(
        num_scalar_prefetch=1,
        grid=(1,),
        in_specs=[
            pl.BlockSpec(memory_space=pltpu.MemorySpace.HBM),
            pl.BlockSpec(memory_space=pltpu.MemorySpace.HBM),
        ],
        out_specs=pl.BlockSpec(memory_space=pltpu.MemorySpace.HBM),
        scratch_shapes=[pltpu.SemaphoreType.DMA((4,))],
    )
    return pl.pallas_call(
        _body,
        grid_spec=grid_spec,
        out_shape=jax.ShapeDtypeStruct(cqt.shape, cqt.dtype),
    )(idx, cqt, pos_cqt)


# speedup vs baseline: 9.4352x; 9.4352x over previous
import functools

import jax
import jax.numpy as jnp
from jax import lax
from jax.experimental import pallas as pl
from jax.experimental.pallas import tpu as pltpu

_GB = 8


def _body(idx_ref, cqt_ref, pos_ref, out_ref, buf0, buf1, g0, g1, w0, w1):
    B, C, F, T = cqt_ref.shape
    n_groups = B // _GB
    bufs = (buf0, buf1)
    gsems = (g0, g1)
    wsems = (w0, w1)

    def issue_reads(g):
        buf, sem = bufs[g % 2], gsems[g % 2]

        def one_batch(i, _):
            b = g * _GB + i
            sel = idx_ref[b]
            for c in range(C):
                @pl.when(sel == c)
                def _():
                    pltpu.make_async_copy(pos_ref.at[b, c], buf.at[i, c], sem).start()

                @pl.when(sel != c)
                def _():
                    pltpu.make_async_copy(cqt_ref.at[b, c], buf.at[i, c], sem).start()

            return 0

        lax.fori_loop(0, _GB, one_batch, 0)

    def wait_reads(g):
        pltpu.make_async_copy(
            cqt_ref.at[pl.ds(g * _GB, _GB)], bufs[g % 2], gsems[g % 2]
        ).wait()

    def start_write(g):
        pltpu.make_async_copy(
            bufs[g % 2], out_ref.at[pl.ds(g * _GB, _GB)], wsems[g % 2]
        ).start()

    def wait_write(g):
        pltpu.make_async_copy(
            bufs[g % 2], out_ref.at[pl.ds(g * _GB, _GB)], wsems[g % 2]
        ).wait()

    issue_reads(0)
    for g in range(n_groups):
        if g >= 1:
            wait_write(g - 1)
        if g + 1 < n_groups:
            issue_reads(g + 1)
        wait_reads(g)
        start_write(g)
    wait_write(n_groups - 1)


def kernel(cqt, pos_cqt, channel_idx):
    B, C, F, T = cqt.shape
    idx = channel_idx.astype(jnp.int32)

    grid_spec = pltpu.PrefetchScalarGridSpec(
        num_scalar_prefetch=1,
        grid=(1,),
        in_specs=[
            pl.BlockSpec(memory_space=pltpu.MemorySpace.HBM),
            pl.BlockSpec(memory_space=pltpu.MemorySpace.HBM),
        ],
        out_specs=pl.BlockSpec(memory_space=pltpu.MemorySpace.HBM),
        scratch_shapes=[
            pltpu.MemorySpace.VMEM((_GB, C, F, T), jnp.float32),
            pltpu.MemorySpace.VMEM((_GB, C, F, T), jnp.float32),
            pltpu.SemaphoreType.DMA,
            pltpu.SemaphoreType.DMA,
            pltpu.SemaphoreType.DMA,
            pltpu.SemaphoreType.DMA,
        ],
    )
    return pl.pallas_call(
        _body,
        grid_spec=grid_spec,
        out_shape=jax.ShapeDtypeStruct(cqt.shape, cqt.dtype),
    )(idx, cqt, pos_cqt)


# speedup vs baseline: 9.5027x; 1.0072x over previous
import functools

import jax
import jax.numpy as jnp
from jax import lax
from jax.experimental import pallas as pl
from jax.experimental.pallas import tpu as pltpu

_GB = 16


def _body(idx_ref, cqt_ref, pos_ref, out_ref, buf0, buf1, g0, g1, w0, w1):
    B, C, F, T = cqt_ref.shape
    n_groups = B // _GB
    bufs = (buf0, buf1)
    gsems = (g0, g1)
    wsems = (w0, w1)

    def issue_reads(g):
        buf, sem = bufs[g % 2], gsems[g % 2]

        for i in range(_GB):
            b = g * _GB + i
            sel = idx_ref[b]
            for c in range(C):
                @pl.when(sel == c)
                def _():
                    pltpu.make_async_copy(pos_ref.at[b, c], buf.at[i, c], sem).start()

                @pl.when(sel != c)
                def _():
                    pltpu.make_async_copy(cqt_ref.at[b, c], buf.at[i, c], sem).start()

    def wait_reads(g):
        pltpu.make_async_copy(
            cqt_ref.at[pl.ds(g * _GB, _GB)], bufs[g % 2], gsems[g % 2]
        ).wait()

    def start_write(g):
        pltpu.make_async_copy(
            bufs[g % 2], out_ref.at[pl.ds(g * _GB, _GB)], wsems[g % 2]
        ).start()

    def wait_write(g):
        pltpu.make_async_copy(
            bufs[g % 2], out_ref.at[pl.ds(g * _GB, _GB)], wsems[g % 2]
        ).wait()

    issue_reads(0)
    for g in range(n_groups):
        if g >= 1:
            wait_write(g - 1)
        if g + 1 < n_groups:
            issue_reads(g + 1)
        wait_reads(g)
        start_write(g)
    wait_write(n_groups - 1)


def kernel(cqt, pos_cqt, channel_idx):
    B, C, F, T = cqt.shape
    idx = channel_idx.astype(jnp.int32)

    grid_spec = pltpu.PrefetchScalarGridSpec(
        num_scalar_prefetch=1,
        grid=(1,),
        in_specs=[
            pl.BlockSpec(memory_space=pltpu.MemorySpace.HBM),
            pl.BlockSpec(memory_space=pltpu.MemorySpace.HBM),
        ],
        out_specs=pl.BlockSpec(memory_space=pltpu.MemorySpace.HBM),
        scratch_shapes=[
            pltpu.MemorySpace.VMEM((_GB, C, F, T), jnp.float32),
            pltpu.MemorySpace.VMEM((_GB, C, F, T), jnp.float32),
            pltpu.SemaphoreType.DMA,
            pltpu.SemaphoreType.DMA,
            pltpu.SemaphoreType.DMA,
            pltpu.SemaphoreType.DMA,
        ],
    )
    return pl.pallas_call(
        _body,
        grid_spec=grid_spec,
        out_shape=jax.ShapeDtypeStruct(cqt.shape, cqt.dtype),
    )(idx, cqt, pos_cqt)
